# baseline (speedup 1.0000x reference)
import jax
import jax.numpy as jnp
from jax.experimental import pallas as pl
from jax.experimental.pallas import tpu as pltpu


def _make_kernel(w, k, fac, bh, c):
    def body(f_ref, cent_ref, out_ref, a_ref, c2_ref, sel_ref):
        i = pl.program_id(0)
        j = pl.program_id(1)

        @pl.when(jnp.logical_and(i == 0, j == 0))
        def _build_constants():
            cent = cent_ref[...]
            crep = jnp.broadcast_to(cent[:, None, :], (k, bh, c))
            crep = crep.reshape(k * bh, c)
            colc = jax.lax.broadcasted_iota(jnp.int32, (c, c * bh), 1)
            rowc = jax.lax.broadcasted_iota(jnp.int32, (c, c * bh), 0)
            selc = (colc // bh == rowc).astype(jnp.float32)
            tmp = jnp.dot(crep, selc,
                          precision=jax.lax.Precision.HIGHEST)
            cola = jax.lax.broadcasted_iota(jnp.int32, (k * bh, c * bh), 1)
            rowa = jax.lax.broadcasted_iota(jnp.int32, (k * bh, c * bh), 0)
            a = jnp.where((cola & (bh - 1)) == (rowa & (bh - 1)), tmp, 0.0)
            a_ref[...] = a
            c2_ref[...] = jnp.sum(a * a, axis=1, keepdims=True)
            colu = jax.lax.broadcasted_iota(jnp.int32, (w, w * fac), 1)
            rowu = jax.lax.broadcasted_iota(jnp.int32, (w, w * fac), 0)
            sel_ref[...] = (colu // fac == rowu).astype(jnp.float32)

        f2d = f_ref[0].reshape(c * bh, w)
        cross = jnp.dot(a_ref[...], f2d)
        score = c2_ref[...] - 2.0 * cross
        s3 = score.reshape(k, bh, w)
        smin = jnp.min(s3, axis=0, keepdims=True)
        kid = jax.lax.broadcasted_iota(jnp.int32, (k, bh, w), 0)
        m = jnp.min(jnp.where(s3 == smin, kid, k), axis=0)
        mf = m.astype(jnp.float32)
        rep = jnp.dot(mf, sel_ref[...]).astype(jnp.int32)
        rep3 = jnp.broadcast_to(rep[:, None, :], (bh, fac, w * fac))
        out_ref[0] = rep3.reshape(bh * fac, w * fac)
    return body


def _assign_and_upsample(feature, centroid, H, W):
    b, c, h, w = feature.shape
    k = centroid.shape[0]
    fac = H // h
    assert H == h * fac and W == w * fac
    bh = 8
    return pl.pallas_call(
        _make_kernel(w, k, fac, bh, c),
        grid=(b, h // bh),
        in_specs=[
            pl.BlockSpec((1, c, bh, w), lambda i, j: (i, 0, j, 0)),
            pl.BlockSpec((k, c), lambda i, j: (0, 0)),
        ],
        out_specs=pl.BlockSpec((1, bh * fac, w * fac), lambda i, j: (i, j, 0)),
        out_shape=jax.ShapeDtypeStruct((b, H, W), jnp.int32),
        scratch_shapes=[
            pltpu.VMEM((k * bh, c * bh), jnp.float32),
            pltpu.VMEM((k * bh, 1), jnp.float32),
            pltpu.VMEM((w, w * fac), jnp.float32),
        ],
    )(feature, centroid)


def kernel(feature_s2t, feature_target, seg_s2t, seg_target, centroid_convert, centroid_target):
    H1, W1 = seg_s2t.shape[1], seg_s2t.shape[2]
    H2, W2 = seg_target.shape[1], seg_target.shape[2]
    mask_s2t_target = _assign_and_upsample(feature_s2t, centroid_target, H1, W1)
    mask_target_s2t = _assign_and_upsample(feature_target, centroid_convert, H2, W2)
    return (mask_s2t_target, mask_target_s2t)


# speedup vs baseline: 1.5956x; 1.5956x over previous
import jax
import jax.numpy as jnp
from jax.experimental import pallas as pl
from jax.experimental.pallas import tpu as pltpu


def _make_kernel(w, k, fac, bh, c):
    def body(f_ref, cent_ref, out_ref, a_ref, c2_ref, sel_ref):
        i = pl.program_id(0)

        @pl.when(i == 0)
        def _build_constants():
            cent = cent_ref[...]
            crep = jnp.broadcast_to(cent[:, None, :], (k, bh, c))
            crep = crep.reshape(k * bh, c)
            colc = jax.lax.broadcasted_iota(jnp.int32, (c, c * bh), 1)
            rowc = jax.lax.broadcasted_iota(jnp.int32, (c, c * bh), 0)
            selc = (colc // bh == rowc).astype(jnp.float32)
            tmp = jnp.dot(crep, selc,
                          precision=jax.lax.Precision.HIGHEST)
            cola = jax.lax.broadcasted_iota(jnp.int32, (k * bh, c * bh), 1)
            rowa = jax.lax.broadcasted_iota(jnp.int32, (k * bh, c * bh), 0)
            a = jnp.where((cola & (bh - 1)) == (rowa & (bh - 1)), tmp, 0.0)
            a_ref[...] = a
            c2_ref[...] = jnp.sum(a * a, axis=1, keepdims=True)
            colu = jax.lax.broadcasted_iota(jnp.int32, (w, w * fac), 1)
            rowu = jax.lax.broadcasted_iota(jnp.int32, (w, w * fac), 0)
            sel_ref[...] = (colu // fac == rowu).astype(jnp.float32)

        f3 = f_ref[0]
        h = f3.shape[1]
        kid = jax.lax.broadcasted_iota(jnp.int32, (k, bh, w), 0)
        for g in range(h // bh):
            f2d = f3[:, g * bh:(g + 1) * bh, :].reshape(c * bh, w)
            cross = jnp.dot(a_ref[...], f2d)
            score = c2_ref[...] - 2.0 * cross
            s3 = score.reshape(k, bh, w)
            smin = jnp.min(s3, axis=0, keepdims=True)
            m = jnp.min(jnp.where(s3 == smin, kid, k), axis=0)
            mf = m.astype(jnp.float32)
            rep = jnp.dot(mf, sel_ref[...]).astype(jnp.int32)
            rep3 = jnp.broadcast_to(rep[:, None, :], (bh, fac, w * fac))
            out_ref[0, g * bh * fac:(g + 1) * bh * fac, :] = (
                rep3.reshape(bh * fac, w * fac))
    return body


def _assign_and_upsample(feature, centroid, H, W):
    b, c, h, w = feature.shape
    k = centroid.shape[0]
    fac = H // h
    assert H == h * fac and W == w * fac
    bh = 8
    return pl.pallas_call(
        _make_kernel(w, k, fac, bh, c),
        grid=(b,),
        in_specs=[
            pl.BlockSpec((1, c, h, w), lambda i: (i, 0, 0, 0)),
            pl.BlockSpec((k, c), lambda i: (0, 0)),
        ],
        out_specs=pl.BlockSpec((1, H, W), lambda i: (i, 0, 0)),
        out_shape=jax.ShapeDtypeStruct((b, H, W), jnp.int32),
        scratch_shapes=[
            pltpu.VMEM((k * bh, c * bh), jnp.float32),
            pltpu.VMEM((k * bh, 1), jnp.float32),
            pltpu.VMEM((w, w * fac), jnp.float32),
        ],
    )(feature, centroid)


def kernel(feature_s2t, feature_target, seg_s2t, seg_target, centroid_convert, centroid_target):
    H1, W1 = seg_s2t.shape[1], seg_s2t.shape[2]
    H2, W2 = seg_target.shape[1], seg_target.shape[2]
    mask_s2t_target = _assign_and_upsample(feature_s2t, centroid_target, H1, W1)
    mask_target_s2t = _assign_and_upsample(feature_target, centroid_convert, H2, W2)
    return (mask_s2t_target, mask_target_s2t)


# speedup vs baseline: 1.7854x; 1.1190x over previous
import jax
import jax.numpy as jnp
from jax.experimental import pallas as pl
from jax.experimental.pallas import tpu as pltpu


def _build_a(cent, k, c, bh):
    crep = jnp.broadcast_to(cent[:, None, :], (k, bh, c))
    crep = crep.reshape(k * bh, c)
    colc = jax.lax.broadcasted_iota(jnp.int32, (c, c * bh), 1)
    rowc = jax.lax.broadcasted_iota(jnp.int32, (c, c * bh), 0)
    selc = (colc // bh == rowc).astype(jnp.float32)
    tmp = jnp.dot(crep, selc,
                  precision=jax.lax.Precision.HIGHEST)
    cola = jax.lax.broadcasted_iota(jnp.int32, (k * bh, c * bh), 1)
    rowa = jax.lax.broadcasted_iota(jnp.int32, (k * bh, c * bh), 0)
    return jnp.where((cola & (bh - 1)) == (rowa & (bh - 1)), tmp, 0.0)


def _make_kernel(w, k, fac, bh, c, h):
    def one_map(f3, a_ref, c2_ref, sel_ref, out_ref, kid):
        for g in range(h // bh):
            f2d = f3[:, g * bh:(g + 1) * bh, :].reshape(c * bh, w)
            cross = jnp.dot(a_ref[...], f2d)
            score = c2_ref[...] - 2.0 * cross
            s3 = score.reshape(k, bh, w)
            smin = jnp.min(s3, axis=0, keepdims=True)
            m = jnp.min(jnp.where(s3 == smin, kid, k), axis=0)
            mf = m.astype(jnp.float32)
            rep = jnp.dot(mf, sel_ref[...]).astype(jnp.int32)
            rep3 = jnp.broadcast_to(rep[:, None, :], (bh, fac, w * fac))
            out_ref[0, g * bh * fac:(g + 1) * bh * fac, :] = (
                rep3.reshape(bh * fac, w * fac))

    def body(f1_ref, f2_ref, cent1_ref, cent2_ref, out1_ref, out2_ref,
             a1_ref, a2_ref, c21_ref, c22_ref, sel_ref):
        i = pl.program_id(0)

        @pl.when(i == 0)
        def _build_constants():
            a1 = _build_a(cent1_ref[...], k, c, bh)
            a1_ref[...] = a1
            c21_ref[...] = jnp.sum(a1 * a1, axis=1, keepdims=True)
            a2 = _build_a(cent2_ref[...], k, c, bh)
            a2_ref[...] = a2
            c22_ref[...] = jnp.sum(a2 * a2, axis=1, keepdims=True)
            colu = jax.lax.broadcasted_iota(jnp.int32, (w, w * fac), 1)
            rowu = jax.lax.broadcasted_iota(jnp.int32, (w, w * fac), 0)
            sel_ref[...] = (colu // fac == rowu).astype(jnp.float32)

        kid = jax.lax.broadcasted_iota(jnp.int32, (k, bh, w), 0)
        one_map(f1_ref[0], a1_ref, c21_ref, sel_ref, out1_ref, kid)
        one_map(f2_ref[0], a2_ref, c22_ref, sel_ref, out2_ref, kid)
    return body


def kernel(feature_s2t, feature_target, seg_s2t, seg_target, centroid_convert, centroid_target):
    b, c, h, w = feature_s2t.shape
    k = centroid_target.shape[0]
    H, W = seg_s2t.shape[1], seg_s2t.shape[2]
    fac = H // h
    assert H == h * fac and W == w * fac
    assert feature_target.shape == (b, c, h, w)
    assert seg_target.shape[1:] == (H, W)
    bh = 8
    out_sds = jax.ShapeDtypeStruct((b, H, W), jnp.int32)
    mask1, mask2 = pl.pallas_call(
        _make_kernel(w, k, fac, bh, c, h),
        grid=(b,),
        in_specs=[
            pl.BlockSpec((1, c, h, w), lambda i: (i, 0, 0, 0)),
            pl.BlockSpec((1, c, h, w), lambda i: (i, 0, 0, 0)),
            pl.BlockSpec((k, c), lambda i: (0, 0)),
            pl.BlockSpec((k, c), lambda i: (0, 0)),
        ],
        out_specs=[
            pl.BlockSpec((1, H, W), lambda i: (i, 0, 0)),
            pl.BlockSpec((1, H, W), lambda i: (i, 0, 0)),
        ],
        out_shape=[out_sds, out_sds],
        scratch_shapes=[
            pltpu.VMEM((k * bh, c * bh), jnp.float32),
            pltpu.VMEM((k * bh, c * bh), jnp.float32),
            pltpu.VMEM((k * bh, 1), jnp.float32),
            pltpu.VMEM((k * bh, 1), jnp.float32),
            pltpu.VMEM((w, w * fac), jnp.float32),
        ],
    )(feature_s2t, feature_target, centroid_target, centroid_convert)
    return (mask1, mask2)
